# Initial kernel scaffold; baseline (speedup 1.0000x reference)
import jax
import jax.numpy as jnp
from jax.experimental import pallas as pl


def kernel(nodes, coord, edges, edge_attr, node_attr, batch, size, params):
    raise NotImplementedError("write your pallas kernel here")


# speedup vs baseline: 3.6919x; 3.6919x over previous
import functools

import jax
import jax.numpy as jnp
from jax import lax
from jax.experimental import pallas as pl
from jax.experimental.pallas import tpu as pltpu
from jax.experimental.pallas import tpu_sc as plsc

NC = 2
NS = 16
NW = NC * NS
CHUNK = 128
W_TAB = 48
SIZE = 50


def _silu(x):
    return x * jax.nn.sigmoid(x)


def _tc_pre(nodes, coord, emb_w, emb_b, w_hr, w_hc):
    n, _ = nodes.shape
    blk = 1000
    grid = (n // blk,)

    def body(nodes_ref, coord_ref, embw_ref, embb_ref, whr_ref, whc_ref,
             h_ref, trow_ref, tcol_ref):
        x = nodes_ref[...] @ embw_ref[...] + embb_ref[...]
        h = _silu(x)
        h_ref[...] = h
        c = coord_ref[...]
        z = jnp.zeros((h.shape[0], W_TAB - 35), jnp.float32)
        trow_ref[...] = jnp.concatenate([h @ whr_ref[...], c, z], axis=1)
        tcol_ref[...] = jnp.concatenate([h @ whc_ref[...], c, z], axis=1)

    full = lambda a: pl.BlockSpec(a.shape, lambda i: (0,) * a.ndim)
    return pl.pallas_call(
        body,
        grid=grid,
        in_specs=[
            pl.BlockSpec((blk, nodes.shape[1]), lambda i: (i, 0)),
            pl.BlockSpec((blk, 3), lambda i: (i, 0)),
            full(emb_w), full(emb_b), full(w_hr), full(w_hc),
        ],
        out_specs=[
            pl.BlockSpec((blk, 32), lambda i: (i, 0)),
            pl.BlockSpec((blk, W_TAB), lambda i: (i, 0)),
            pl.BlockSpec((blk, W_TAB), lambda i: (i, 0)),
        ],
        out_shape=[
            jax.ShapeDtypeStruct((n, 32), jnp.float32),
            jax.ShapeDtypeStruct((n, W_TAB), jnp.float32),
            jax.ShapeDtypeStruct((n, W_TAB), jnp.float32),
        ],
    )(nodes, coord, emb_w, emb_b, w_hr, w_hc)


def _sc_gather(trow, tcol, rowi, coli, e_pad, cpw):
    epw = cpw * CHUNK
    mesh = plsc.VectorSubcoreMesh(
        core_axis_name="c", subcore_axis_name="s",
        num_cores=NC, num_subcores=NS)

    @functools.partial(
        pl.kernel,
        out_type=(jax.ShapeDtypeStruct((e_pad, W_TAB), jnp.float32),
                  jax.ShapeDtypeStruct((e_pad, W_TAB), jnp.float32)),
        mesh=mesh,
        scratch_types=[
            pltpu.VMEM((cpw, CHUNK), jnp.int32),
            pltpu.VMEM((cpw, CHUNK), jnp.int32),
            pltpu.VMEM((CHUNK, W_TAB), jnp.float32),
            pltpu.VMEM((CHUNK, W_TAB), jnp.float32),
            pltpu.SemaphoreType.DMA,
            pltpu.SemaphoreType.DMA,
        ],
        compiler_params=pltpu.CompilerParams(use_tc_tiling_on_sc=False),
    )
    def gather_k(trow_hbm, tcol_hbm, rowi_hbm, coli_hbm, grow_hbm, gcol_hbm,
                 rv, cv, bufa, bufb, sa, sb):
        c = lax.axis_index("c")
        s = lax.axis_index("s")
        wid = s * NC + c
        base = wid * epw
        pltpu.sync_copy(rowi_hbm.at[wid], rv)
        pltpu.sync_copy(coli_hbm.at[wid], cv)

        def body(j, carry):
            ca = pltpu.async_copy(trow_hbm.at[rv.at[j]], bufa, sa)
            cb = pltpu.async_copy(tcol_hbm.at[cv.at[j]], bufb, sb)
            ca.wait()
            cb.wait()
            pltpu.sync_copy(bufa, grow_hbm.at[pl.ds(base + j * CHUNK, CHUNK)])
            pltpu.sync_copy(bufb, gcol_hbm.at[pl.ds(base + j * CHUNK, CHUNK)])
            return carry

        lax.fori_loop(0, cpw, body, 0)

    return gather_k(trow, tcol, rowi, coli)


def _tc_edge(grow, gcol, ea, w_r, w_ea, b1, e2w, e2b, a1w, a1b, a2w, a2b):
    e_pad = grow.shape[0]
    blk = 2048
    grid = (e_pad // blk,)

    def body(gr_ref, gc_ref, ea_ref, wr_ref, wea_ref, b1_ref,
             e2w_ref, e2b_ref, a1w_ref, a1b_ref, a2w_ref, a2b_ref, ef_ref):
        gr = gr_ref[...]
        gc = gc_ref[...]
        d = gr[:, 32:] - gc[:, 32:]
        radial = jnp.sum(d * d, axis=1, keepdims=True)
        x = (gr[:, :32] + gc[:, :32] + radial * wr_ref[...]
             + ea_ref[...] @ wea_ref[...] + b1_ref[...])
        m = _silu(x)
        m = _silu(m @ e2w_ref[...] + e2b_ref[...])
        att = jax.nn.sigmoid(
            _silu(m @ a1w_ref[...] + a1b_ref[...]) @ a2w_ref[...] + a2b_ref[...])
        ef_ref[...] = m * att

    full = lambda a: pl.BlockSpec(a.shape, lambda i: (0,) * a.ndim)
    return pl.pallas_call(
        body,
        grid=grid,
        in_specs=[
            pl.BlockSpec((blk, W_TAB), lambda i: (i, 0)),
            pl.BlockSpec((blk, W_TAB), lambda i: (i, 0)),
            pl.BlockSpec((blk, 16), lambda i: (i, 0)),
            full(w_r), full(w_ea), full(b1),
            full(e2w), full(e2b), full(a1w), full(a1b), full(a2w), full(a2b),
        ],
        out_specs=pl.BlockSpec((blk, 32), lambda i: (i, 0)),
        out_shape=jax.ShapeDtypeStruct((e_pad, 32), jnp.float32),
    )(grow, gcol, ea, w_r, w_ea, b1, e2w, e2b, a1w, a1b, a2w, a2b)


def _sc_scatter(ef, rowi_s, zeros_hbm, nacc, cpw):
    epw = cpw * CHUNK
    rps = nacc // NS
    mesh = plsc.VectorSubcoreMesh(
        core_axis_name="c", subcore_axis_name="s",
        num_cores=NC, num_subcores=NS)

    @functools.partial(
        pl.kernel,
        out_type=jax.ShapeDtypeStruct((NC, nacc, 32), jnp.float32),
        mesh=mesh,
        scratch_types=[
            pltpu.VMEM((cpw, CHUNK), jnp.int32),
            pltpu.VMEM((CHUNK, 32), jnp.float32),
            pltpu.VMEM_SHARED((nacc, 32), jnp.float32),
        ],
        compiler_params=pltpu.CompilerParams(use_tc_tiling_on_sc=False),
    )
    def scatter_k(ef_hbm, rowi_hbm, z_hbm, out_hbm, idxv, efv, acc):
        c = lax.axis_index("c")
        s = lax.axis_index("s")
        wid = s * NC + c
        base = wid * epw
        pltpu.sync_copy(z_hbm, acc.at[pl.ds(s * rps, rps)])
        plsc.subcore_barrier()
        pltpu.sync_copy(rowi_hbm.at[wid], idxv)

        def body(j, carry):
            pltpu.sync_copy(ef_hbm.at[pl.ds(base + j * CHUNK, CHUNK)], efv)
            pltpu.sync_copy(efv, acc.at[idxv.at[j]], add=True)
            return carry

        lax.fori_loop(0, cpw, body, 0)
        plsc.subcore_barrier()
        pltpu.sync_copy(acc.at[pl.ds(s * rps, rps)],
                        out_hbm.at[c].at[pl.ds(s * rps, rps)])

    return scatter_k(ef, rowi_s, zeros_hbm)


def _tc_node(h, agg2, node_attr, seg3, n1w, n1b, n2w, n2b,
             en1w, en1b, en2w, en2b):
    n = h.shape[0]
    blk = 1000
    grid = (n // blk,)

    def body(h_ref, agg_ref, na_ref, seg_ref, n1w_ref, n1b_ref, n2w_ref,
             n2b_ref, en1w_ref, en1b_ref, en2w_ref, en2b_ref, pooled_ref):
        i = pl.program_id(0)
        h = h_ref[...]
        agg = agg_ref[0] + agg_ref[1]
        nin = jnp.concatenate([h, agg, na_ref[...]], axis=1)
        t = _silu(nin @ n1w_ref[...] + n1b_ref[...])
        h2 = h + t @ n2w_ref[...] + n2b_ref[...]
        h3 = _silu(h2 @ en1w_ref[...] + en1b_ref[...]) @ en2w_ref[...] + en2b_ref[...]
        seg = seg_ref[0, 0, :]
        oh = (lax.broadcasted_iota(jnp.int32, (SIZE, blk), 0)
              == seg[None, :]).astype(jnp.float32)

        @pl.when(i == 0)
        def _():
            pooled_ref[...] = jnp.zeros_like(pooled_ref)

        pooled_ref[...] += oh @ h3

    full = lambda a: pl.BlockSpec(a.shape, lambda i: (0,) * a.ndim)
    return pl.pallas_call(
        body,
        grid=grid,
        in_specs=[
            pl.BlockSpec((blk, 32), lambda i: (i, 0)),
            pl.BlockSpec((2, blk, 32), lambda i: (0, i, 0)),
            pl.BlockSpec((blk, 16), lambda i: (i, 0)),
            pl.BlockSpec((1, 1, blk), lambda i: (i, 0, 0)),
            full(n1w), full(n1b), full(n2w), full(n2b),
            full(en1w), full(en1b), full(en2w), full(en2b),
        ],
        out_specs=pl.BlockSpec((SIZE, 32), lambda i: (0, 0)),
        out_shape=jax.ShapeDtypeStruct((SIZE, 32), jnp.float32),
    )(h, agg2, node_attr, seg3, n1w, n1b, n2w, n2b, en1w, en1b, en2w, en2b)


def _tc_decode(pooled, d1w, d1b, d2w, d2b):
    def body(p_ref, d1w_ref, d1b_ref, d2w_ref, d2b_ref, out_ref):
        t = _silu(p_ref[...] @ d1w_ref[...] + d1b_ref[...])
        out_ref[...] = t @ d2w_ref[...] + d2b_ref[...]

    return pl.pallas_call(
        body,
        out_shape=jax.ShapeDtypeStruct((SIZE, 1), jnp.float32),
    )(pooled, d1w, d1b, d2w, d2b)


def kernel(nodes, coord, edges, edge_attr, node_attr, batch, size, params):
    p = params
    n = nodes.shape[0]
    e = edges.shape[1]
    row = edges[0].astype(jnp.int32)
    col = edges[1].astype(jnp.int32)

    cpw = -(-e // (NW * CHUNK))
    e_pad = NW * cpw * CHUNK
    pad = e_pad - e

    nacc = ((n + 1 + NS * 8 - 1) // (NS * 8)) * (NS * 8)

    row_g = jnp.concatenate([row, jnp.zeros((pad,), jnp.int32)])
    col_g = jnp.concatenate([col, jnp.zeros((pad,), jnp.int32)])
    row_s = jnp.concatenate([row, jnp.full((pad,), n, jnp.int32)])
    rowi = row_g.reshape(NW, cpw, CHUNK)
    coli = col_g.reshape(NW, cpw, CHUNK)
    rowi_s = row_s.reshape(NW, cpw, CHUNK)
    ea_pad = jnp.concatenate(
        [edge_attr, jnp.zeros((pad, edge_attr.shape[1]), jnp.float32)])

    e1w = p['e1_w']
    w_hr = e1w[0:32]
    w_hc = e1w[32:64]
    w_r = e1w[64:65]
    w_ea = e1w[65:81]
    r2 = lambda b: b.reshape(1, -1)

    h, trow, tcol = _tc_pre(nodes, coord, p['emb_w'], r2(p['emb_b']),
                            w_hr, w_hc)
    grow, gcol = _sc_gather(trow, tcol, rowi, coli, e_pad, cpw)
    ef = _tc_edge(grow, gcol, ea_pad, w_r, w_ea, r2(p['e1_b']),
                  p['e2_w'], r2(p['e2_b']), p['a1_w'], r2(p['a1_b']),
                  p['a2_w'], r2(p['a2_b']))
    zeros_hbm = jnp.zeros((nacc // NS, 32), jnp.float32)
    agg2 = _sc_scatter(ef, rowi_s, zeros_hbm, nacc, cpw)

    seg = jnp.minimum(batch, size - 1).astype(jnp.int32)
    seg3 = seg.reshape(SIZE, 1, n // SIZE)

    pooled = _tc_node(h, agg2, node_attr, seg3,
                      p['n1_w'], r2(p['n1_b']), p['n2_w'], r2(p['n2_b']),
                      p['en1_w'], r2(p['en1_b']), p['en2_w'], r2(p['en2_b']))
    return _tc_decode(pooled, p['d1_w'], r2(p['d1_b']),
                      p['d2_w'], r2(p['d2_b']))
